# baseline (speedup 1.0000x reference)
import jax
import jax.numpy as jnp
from jax import lax
from jax.experimental import pallas as pl
from jax.experimental.pallas import tpu as pltpu
from jax.experimental.pallas import tpu_sc as plsc

_N = 10000
_E = 320000
_D = 128
_B = 64
_NP = 10240
_BM = 1024
_NC, _NS = 2, 16
_NW = _NC * _NS
_EW = _E // _NW
_K = 80
_CH = _EW // _K
_RT = _NP // _NS

_sc_mesh = plsc.VectorSubcoreMesh(core_axis_name="c", subcore_axis_name="s")


def _deg_body(dst_hbm, ones_hbm, zeros_hbm, out_hbm,
              didx_a, didx_b, rows_ones, isem_a, isem_b, ssem_a, ssem_b,
              accum):
    cc = lax.axis_index("c")
    s = lax.axis_index("s")
    base = (cc * _NS + s) * _EW

    def iload(chunk, didx, isem):
        pltpu.async_copy(dst_hbm.at[pl.ds(base + chunk * _K, _K)], didx, isem)

    def iwait(chunk, didx, isem):
        pltpu.make_async_copy(dst_hbm.at[pl.ds(base + chunk * _K, _K)],
                              didx, isem).wait()

    iload(0, didx_a, isem_a)
    iload(1, didx_b, isem_b)
    pltpu.sync_copy(ones_hbm, rows_ones)
    pltpu.sync_copy(zeros_hbm, accum.at[pl.ds(s * _RT, _RT)])
    plsc.subcore_barrier()

    def pair(i, carry):
        c0 = 2 * i
        iwait(c0, didx_a, isem_a)
        pltpu.async_copy(rows_ones, accum.at[didx_a], ssem_a, add=True)
        iwait(c0 + 1, didx_b, isem_b)
        pltpu.async_copy(rows_ones, accum.at[didx_b], ssem_b, add=True)
        pltpu.make_async_copy(rows_ones, accum.at[didx_a], ssem_a).wait()
        iload(c0 + 2, didx_a, isem_a)
        pltpu.make_async_copy(rows_ones, accum.at[didx_b], ssem_b).wait()
        iload(c0 + 3, didx_b, isem_b)
        return carry

    lax.fori_loop(0, (_CH - 1) // 2, pair, 0)
    iwait(_CH - 1, didx_a, isem_a)
    pltpu.sync_copy(rows_ones, accum.at[didx_a], add=True)
    iwait(_CH, didx_b, isem_b)
    plsc.subcore_barrier()
    pltpu.sync_copy(accum.at[pl.ds(s * _RT, _RT)],
                    out_hbm.at[pl.ds(cc * _NP + s * _RT, _RT)])


def _deg_call(dstp, ones_rows, zeros128):
    return pl.kernel(
        _deg_body,
        out_type=jax.ShapeDtypeStruct((2 * _NP, _D), jnp.float32),
        mesh=_sc_mesh,
        scratch_types=[
            pltpu.VMEM((_K,), jnp.int32),
            pltpu.VMEM((_K,), jnp.int32),
            pltpu.VMEM((_K, _D), jnp.float32),
            pltpu.SemaphoreType.DMA,
            pltpu.SemaphoreType.DMA,
            pltpu.SemaphoreType.DMA,
            pltpu.SemaphoreType.DMA,
            pltpu.VMEM_SHARED((_NP, _D), jnp.float32),
        ],
    )(dstp, ones_rows, zeros128)


def _scatter_body(hws_hbm, src_hbm, dst_hbm, zeros_hbm, out_hbm,
                  sidx_a, didx_a, sidx_b, didx_b, rows_a, rows_b,
                  isem_a, isem_b, gsem_a, gsem_b, accum):
    cc = lax.axis_index("c")
    s = lax.axis_index("s")
    base = (cc * _NS + s) * _EW

    def iload(chunk, sidx, didx, isem):
        pltpu.async_copy(src_hbm.at[pl.ds(base + chunk * _K, _K)], sidx, isem)
        pltpu.async_copy(dst_hbm.at[pl.ds(base + chunk * _K, _K)], didx, isem)

    def iwait(chunk, sidx, didx, isem):
        pltpu.make_async_copy(src_hbm.at[pl.ds(base + chunk * _K, _K)],
                              sidx, isem).wait()
        pltpu.make_async_copy(dst_hbm.at[pl.ds(base + chunk * _K, _K)],
                              didx, isem).wait()

    def gstart(sidx, rows, gsem):
        pltpu.async_copy(hws_hbm.at[sidx], rows, gsem)

    def gwait(sidx, rows, gsem):
        pltpu.make_async_copy(hws_hbm.at[sidx], rows, gsem).wait()

    iload(0, sidx_a, didx_a, isem_a)
    pltpu.sync_copy(zeros_hbm, accum.at[pl.ds(s * _RT, _RT)])
    plsc.subcore_barrier()
    iwait(0, sidx_a, didx_a, isem_a)
    gstart(sidx_a, rows_a, gsem_a)
    iload(1, sidx_b, didx_b, isem_b)

    def pair(i, carry):
        c1 = 2 * i + 1
        iwait(c1, sidx_b, didx_b, isem_b)
        gstart(sidx_b, rows_b, gsem_b)
        gwait(sidx_a, rows_a, gsem_a)
        pltpu.sync_copy(rows_a, accum.at[didx_a], add=True)
        iload(c1 + 1, sidx_a, didx_a, isem_a)
        gwait(sidx_b, rows_b, gsem_b)
        pltpu.sync_copy(rows_b, accum.at[didx_b], add=True)
        iwait(c1 + 1, sidx_a, didx_a, isem_a)
        gstart(sidx_a, rows_a, gsem_a)
        iload(c1 + 2, sidx_b, didx_b, isem_b)
        return carry

    lax.fori_loop(0, (_CH - 1) // 2, pair, 0)
    gwait(sidx_a, rows_a, gsem_a)
    pltpu.sync_copy(rows_a, accum.at[didx_a], add=True)
    iwait(_CH, sidx_b, didx_b, isem_b)
    plsc.subcore_barrier()
    pltpu.sync_copy(accum.at[pl.ds(s * _RT, _RT)],
                    out_hbm.at[pl.ds(cc * _NP + s * _RT, _RT)])


def _scatter_call(hws, srcp, dstp, zeros128):
    return pl.kernel(
        _scatter_body,
        out_type=jax.ShapeDtypeStruct((2 * _NP, _D), jnp.float32),
        mesh=_sc_mesh,
        scratch_types=[
            pltpu.VMEM((_K,), jnp.int32),
            pltpu.VMEM((_K,), jnp.int32),
            pltpu.VMEM((_K,), jnp.int32),
            pltpu.VMEM((_K,), jnp.int32),
            pltpu.VMEM((_K, _D), jnp.float32),
            pltpu.VMEM((_K, _D), jnp.float32),
            pltpu.SemaphoreType.DMA,
            pltpu.SemaphoreType.DMA,
            pltpu.SemaphoreType.DMA,
            pltpu.SemaphoreType.DMA,
            pltpu.VMEM_SHARED((_NP, _D), jnp.float32),
        ],
    )(hws, srcp, dstp, zeros128)


_PREC = lax.Precision.HIGHEST


def _dot(a, b):
    return jax.lax.dot_general(a, b, (((a.ndim - 1,), (0,)), ((), ())),
                               precision=_PREC,
                               preferred_element_type=jnp.float32)


def _dinv(d0_ref, d1_ref):
    return 1.0 / jnp.sqrt(d0_ref[...] + d1_ref[...] + 1.0)


def _mm_scale_body(x_ref, w_ref, d0_ref, d1_ref, o_ref):
    o_ref[...] = _dot(x_ref[...], w_ref[...]) * _dinv(d0_ref, d1_ref)


def _mm_scale(x, w, dg):
    return pl.pallas_call(
        _mm_scale_body,
        grid=(_NP // _BM,),
        in_specs=[
            pl.BlockSpec((_BM, _D), lambda i: (i, 0)),
            pl.BlockSpec((_D, _D), lambda i: (0, 0)),
            pl.BlockSpec((_BM, 1), lambda i: (i, 0)),
            pl.BlockSpec((_BM, 1), lambda i: (i + _NP // _BM, 0)),
        ],
        out_specs=pl.BlockSpec((_BM, _D), lambda i: (i, 0)),
        out_shape=jax.ShapeDtypeStruct((_NP, _D), jnp.float32),
    )(x, w, dg, dg)


def _stats_phase(a0_ref, a1_ref, hws_ref, d0_ref, d1_ref, b_ref,
                 s_scr, st_scr, i):
    s = (a0_ref[...] + a1_ref[...] + hws_ref[...]) * _dinv(d0_ref, d1_ref) \
        + b_ref[...]
    s_scr[...] = s
    rows = lax.broadcasted_iota(jnp.int32, (_BM, 1), 0) + i * _BM
    sv = jnp.where(rows < _N, s, 0.0)
    upd = jnp.concatenate(
        [jnp.sum(sv, axis=0, keepdims=True),
         jnp.sum(sv * sv, axis=0, keepdims=True),
         jnp.zeros((6, _D), jnp.float32)], axis=0)

    @pl.when(i == 0)
    def _():
        st_scr[...] = upd

    @pl.when(i > 0)
    def _():
        st_scr[...] += upd


def _bn(s, st_ref, g_ref, be_ref):
    m = st_ref[0:1, :] / _N
    var = st_ref[1:2, :] / _N - m * m
    rstd = 1.0 / jnp.sqrt(var + 1e-5)
    return jnp.maximum((s - m) * rstd * g_ref[...] + be_ref[...], 0.0)


def _layer_mm_body(a0_ref, a1_ref, hws_ref, d0_ref, d1_ref, b_ref,
                   g_ref, be_ref, w_ref, o_ref, s_scr, st_scr):
    p = pl.program_id(0)
    i = pl.program_id(1)

    @pl.when(p == 0)
    def _():
        _stats_phase(a0_ref, a1_ref, hws_ref, d0_ref, d1_ref, b_ref,
                     s_scr.at[i], st_scr, i)

    @pl.when(p == 1)
    def _():
        h = _bn(s_scr[i], st_scr, g_ref, be_ref)
        o_ref[...] = _dot(h, w_ref[...]) * _dinv(d0_ref, d1_ref)


def _layer_mm(ap, hws, dg, b, g, be, w):
    nb = _NP // _BM
    return pl.pallas_call(
        _layer_mm_body,
        grid=(2, nb),
        in_specs=[
            pl.BlockSpec((_BM, _D), lambda p, i: (i, 0)),
            pl.BlockSpec((_BM, _D), lambda p, i: (i + _NP // _BM, 0)),
            pl.BlockSpec((_BM, _D), lambda p, i: (i, 0)),
            pl.BlockSpec((_BM, 1), lambda p, i: (i, 0)),
            pl.BlockSpec((_BM, 1), lambda p, i: (i + _NP // _BM, 0)),
            pl.BlockSpec((1, _D), lambda p, i: (0, 0)),
            pl.BlockSpec((1, _D), lambda p, i: (0, 0)),
            pl.BlockSpec((1, _D), lambda p, i: (0, 0)),
            pl.BlockSpec((_D, _D), lambda p, i: (0, 0)),
        ],
        out_specs=pl.BlockSpec((_BM, _D), lambda p, i: (i, 0)),
        out_shape=jax.ShapeDtypeStruct((_NP, _D), jnp.float32),
        scratch_shapes=[
            pltpu.VMEM((_NP // _BM, _BM, _D), jnp.float32),
            pltpu.VMEM((8, _D), jnp.float32),
        ],
    )(ap, ap, hws, dg, dg, b, g, be, w)


def _layer_pool_body(a0_ref, a1_ref, hws_ref, d0_ref, d1_ref, b_ref,
                     g_ref, be_ref, batch_ref,
                     psum_ref, pmax_ref, pcnt_ref, s_scr, st_scr):
    p = pl.program_id(0)
    i = pl.program_id(1)

    @pl.when(p == 0)
    def _():
        _stats_phase(a0_ref, a1_ref, hws_ref, d0_ref, d1_ref, b_ref,
                     s_scr.at[i], st_scr, i)

    @pl.when(p == 1)
    def _():
        h = _bn(s_scr[i], st_scr, g_ref, be_ref)
        bids = batch_ref[...]
        oh = (bids == lax.broadcasted_iota(jnp.int32, (_BM, _B), 1))
        ohf = oh.astype(jnp.float32)
        dn = (((0,), (0,)), ((), ()))
        psum_upd = lax.dot_general(ohf, h, dn, precision=_PREC,
                                   preferred_element_type=jnp.float32)
        pcnt_upd = lax.dot_general(ohf, jnp.ones((_BM, _D), jnp.float32),
                                   dn, precision=_PREC,
                                   preferred_element_type=jnp.float32)
        neg = jnp.float32(-jnp.inf)
        pmax_upd = jnp.concatenate(
            [jnp.max(jnp.where(bids == b, h, neg), axis=0, keepdims=True)
             for b in range(_B)], axis=0)

        @pl.when(i == 0)
        def _():
            psum_ref[...] = psum_upd
            pmax_ref[...] = pmax_upd
            pcnt_ref[...] = pcnt_upd

        @pl.when(i > 0)
        def _():
            psum_ref[...] += psum_upd
            pmax_ref[...] = jnp.maximum(pmax_ref[...], pmax_upd)
            pcnt_ref[...] += pcnt_upd


def _layer_pool(ap, hws, dg, b, g, be, batch2d):
    nb = _NP // _BM
    return pl.pallas_call(
        _layer_pool_body,
        grid=(2, nb),
        in_specs=[
            pl.BlockSpec((_BM, _D), lambda p, i: (i, 0)),
            pl.BlockSpec((_BM, _D), lambda p, i: (i + _NP // _BM, 0)),
            pl.BlockSpec((_BM, _D), lambda p, i: (i, 0)),
            pl.BlockSpec((_BM, 1), lambda p, i: (i, 0)),
            pl.BlockSpec((_BM, 1), lambda p, i: (i + _NP // _BM, 0)),
            pl.BlockSpec((1, _D), lambda p, i: (0, 0)),
            pl.BlockSpec((1, _D), lambda p, i: (0, 0)),
            pl.BlockSpec((1, _D), lambda p, i: (0, 0)),
            pl.BlockSpec((_BM, 1), lambda p, i: (i, 0)),
        ],
        out_specs=[
            pl.BlockSpec((_B, _D), lambda p, i: (0, 0)),
            pl.BlockSpec((_B, _D), lambda p, i: (0, 0)),
            pl.BlockSpec((_B, _D), lambda p, i: (0, 0)),
        ],
        out_shape=[
            jax.ShapeDtypeStruct((_B, _D), jnp.float32),
            jax.ShapeDtypeStruct((_B, _D), jnp.float32),
            jax.ShapeDtypeStruct((_B, _D), jnp.float32),
        ],
        scratch_shapes=[
            pltpu.VMEM((_NP // _BM, _BM, _D), jnp.float32),
            pltpu.VMEM((8, _D), jnp.float32),
        ],
    )(ap, ap, hws, dg, dg, b, g, be, batch2d)


def _head_body(psum_ref, pmax_ref, pcnt_ref, w1_ref, b1_ref, w2_ref, b2_ref,
               w3_ref, b3_ref, o_ref):
    cnt = pcnt_ref[...]
    mean = psum_ref[...] / jnp.maximum(cnt, 1.0)
    mx = jnp.where(cnt > 0, pmax_ref[...], 0.0)
    z = jnp.concatenate([mean, mx], axis=1)
    z = jnp.maximum(_dot(z, w1_ref[...]) + b1_ref[...], 0.0)
    z = jnp.maximum(_dot(z, w2_ref[...]) + b2_ref[...], 0.0)
    z = _dot(z, w3_ref[...]) + b3_ref[...]
    o_ref[...] = 1.0 / (1.0 + jnp.exp(-z))


def _head(psum, pmax, pcnt, w1, b1, w2, b2, w3, b3):
    return pl.pallas_call(
        _head_body,
        out_shape=jax.ShapeDtypeStruct((_B, 1), jnp.float32),
    )(psum, pmax, pcnt, w1, b1, w2, b2, w3, b3)


def kernel(x, edge_index, batch, W0, b0, g0, be0, W1, b1, g1, be1,
           W2, b2, g2, be2, cW1, cb1, cW2, cb2, cW3, cb3):
    srcp = jnp.pad(edge_index[0], (0, _K))
    dstp = jnp.pad(edge_index[1], (0, _K))
    x_p = jnp.pad(x, ((0, _NP - _N), (0, 0)))
    batch_p = jnp.pad(batch, (0, _NP - _N),
                      constant_values=_B).reshape(_NP, 1)
    zeros128 = jnp.zeros((_RT, _D), jnp.float32)
    ones_rows = jnp.ones((_K, _D), jnp.float32)

    dg_raw = _deg_call(dstp, ones_rows, zeros128)
    dg = dg_raw[:, 0:1]

    hws = _mm_scale(x_p, W0, dg)
    layers = [(b0, g0, be0, W1), (b1, g1, be1, W2), (b2, g2, be2, None)]
    psum = pmax = pcnt = None
    for b, g, be, w_next in layers:
        ap = _scatter_call(hws, srcp, dstp, zeros128)
        if w_next is not None:
            hws = _layer_mm(ap, hws, dg, b.reshape(1, _D), g.reshape(1, _D),
                            be.reshape(1, _D), w_next)
        else:
            psum, pmax, pcnt = _layer_pool(ap, hws, dg, b.reshape(1, _D),
                                           g.reshape(1, _D),
                                           be.reshape(1, _D), batch_p)

    out = _head(psum, pmax, pcnt, cW1, cb1.reshape(1, _D),
                cW2, cb2.reshape(1, _D // 2), cW3, cb3.reshape(1, 1))
    return out.reshape(_B)


# speedup vs baseline: 1.0096x; 1.0096x over previous
import jax
import jax.numpy as jnp
from jax import lax
from jax.experimental import pallas as pl
from jax.experimental.pallas import tpu as pltpu
from jax.experimental.pallas import tpu_sc as plsc

_N = 10000
_E = 320000
_D = 128
_B = 64
_NP = 10240
_BM = 1024
_NC, _NS = 2, 16
_NW = _NC * _NS
_EW = _E // _NW
_K = 80
_CH = _EW // _K
_RT = _NP // _NS

_sc_mesh = plsc.VectorSubcoreMesh(core_axis_name="c", subcore_axis_name="s")


def _deg_body(dst_hbm, ones_hbm, zeros_hbm, out_hbm,
              didx_a, didx_b, rows_ones, isem_a, isem_b, ssem_a, ssem_b,
              accum):
    cc = lax.axis_index("c")
    s = lax.axis_index("s")
    base = (cc * _NS + s) * _EW

    def iload(chunk, didx, isem):
        pltpu.async_copy(dst_hbm.at[pl.ds(base + chunk * _K, _K)], didx, isem)

    def iwait(chunk, didx, isem):
        pltpu.make_async_copy(dst_hbm.at[pl.ds(base + chunk * _K, _K)],
                              didx, isem).wait()

    iload(0, didx_a, isem_a)
    iload(1, didx_b, isem_b)
    pltpu.sync_copy(ones_hbm, rows_ones)
    pltpu.sync_copy(zeros_hbm, accum.at[pl.ds(s * _RT, _RT)])
    plsc.subcore_barrier()

    def pair(i, carry):
        c0 = 2 * i
        iwait(c0, didx_a, isem_a)
        pltpu.async_copy(rows_ones, accum.at[didx_a], ssem_a, add=True)
        iwait(c0 + 1, didx_b, isem_b)
        pltpu.async_copy(rows_ones, accum.at[didx_b], ssem_b, add=True)
        pltpu.make_async_copy(rows_ones, accum.at[didx_a], ssem_a).wait()
        iload(c0 + 2, didx_a, isem_a)
        pltpu.make_async_copy(rows_ones, accum.at[didx_b], ssem_b).wait()
        iload(c0 + 3, didx_b, isem_b)
        return carry

    lax.fori_loop(0, (_CH - 1) // 2, pair, 0)
    iwait(_CH - 1, didx_a, isem_a)
    pltpu.sync_copy(rows_ones, accum.at[didx_a], add=True)
    iwait(_CH, didx_b, isem_b)
    plsc.subcore_barrier()
    pltpu.sync_copy(accum.at[pl.ds(s * _RT, _RT)],
                    out_hbm.at[pl.ds(cc * _NP + s * _RT, _RT)])


def _deg_call(dstp, ones_rows, zeros128):
    return pl.kernel(
        _deg_body,
        out_type=jax.ShapeDtypeStruct((2 * _NP, _D), jnp.float32),
        mesh=_sc_mesh,
        scratch_types=[
            pltpu.VMEM((_K,), jnp.int32),
            pltpu.VMEM((_K,), jnp.int32),
            pltpu.VMEM((_K, _D), jnp.float32),
            pltpu.SemaphoreType.DMA,
            pltpu.SemaphoreType.DMA,
            pltpu.SemaphoreType.DMA,
            pltpu.SemaphoreType.DMA,
            pltpu.VMEM_SHARED((_NP, _D), jnp.float32),
        ],
    )(dstp, ones_rows, zeros128)


def _scatter_body(hws_hbm, src_hbm, dst_hbm, zeros_hbm, out_hbm,
                  sidx_a, didx_a, sidx_b, didx_b, rows_a, rows_b,
                  isem_a, isem_b, gsem_a, gsem_b, accum):
    cc = lax.axis_index("c")
    s = lax.axis_index("s")
    base = (cc * _NS + s) * _EW

    def iload(chunk, sidx, didx, isem):
        pltpu.async_copy(src_hbm.at[pl.ds(base + chunk * _K, _K)], sidx, isem)
        pltpu.async_copy(dst_hbm.at[pl.ds(base + chunk * _K, _K)], didx, isem)

    def iwait(chunk, sidx, didx, isem):
        pltpu.make_async_copy(src_hbm.at[pl.ds(base + chunk * _K, _K)],
                              sidx, isem).wait()
        pltpu.make_async_copy(dst_hbm.at[pl.ds(base + chunk * _K, _K)],
                              didx, isem).wait()

    def gstart(sidx, rows, gsem):
        pltpu.async_copy(hws_hbm.at[sidx], rows, gsem)

    def gwait(sidx, rows, gsem):
        pltpu.make_async_copy(hws_hbm.at[sidx], rows, gsem).wait()

    iload(0, sidx_a, didx_a, isem_a)
    pltpu.sync_copy(zeros_hbm, accum.at[pl.ds(s * _RT, _RT)])
    plsc.subcore_barrier()
    iwait(0, sidx_a, didx_a, isem_a)
    gstart(sidx_a, rows_a, gsem_a)
    iload(1, sidx_b, didx_b, isem_b)

    def pair(i, carry):
        c1 = 2 * i + 1
        iwait(c1, sidx_b, didx_b, isem_b)
        gstart(sidx_b, rows_b, gsem_b)
        gwait(sidx_a, rows_a, gsem_a)
        pltpu.sync_copy(rows_a, accum.at[didx_a], add=True)
        iload(c1 + 1, sidx_a, didx_a, isem_a)
        iwait(c1 + 1, sidx_a, didx_a, isem_a)
        gstart(sidx_a, rows_a, gsem_a)
        gwait(sidx_b, rows_b, gsem_b)
        pltpu.sync_copy(rows_b, accum.at[didx_b], add=True)
        iload(c1 + 2, sidx_b, didx_b, isem_b)
        return carry

    lax.fori_loop(0, (_CH - 1) // 2, pair, 0)
    gwait(sidx_a, rows_a, gsem_a)
    pltpu.sync_copy(rows_a, accum.at[didx_a], add=True)
    iwait(_CH, sidx_b, didx_b, isem_b)
    plsc.subcore_barrier()
    pltpu.sync_copy(accum.at[pl.ds(s * _RT, _RT)],
                    out_hbm.at[pl.ds(cc * _NP + s * _RT, _RT)])


def _scatter_call(hws, srcp, dstp, zeros128):
    return pl.kernel(
        _scatter_body,
        out_type=jax.ShapeDtypeStruct((2 * _NP, _D), jnp.float32),
        mesh=_sc_mesh,
        scratch_types=[
            pltpu.VMEM((_K,), jnp.int32),
            pltpu.VMEM((_K,), jnp.int32),
            pltpu.VMEM((_K,), jnp.int32),
            pltpu.VMEM((_K,), jnp.int32),
            pltpu.VMEM((_K, _D), jnp.float32),
            pltpu.VMEM((_K, _D), jnp.float32),
            pltpu.SemaphoreType.DMA,
            pltpu.SemaphoreType.DMA,
            pltpu.SemaphoreType.DMA,
            pltpu.SemaphoreType.DMA,
            pltpu.VMEM_SHARED((_NP, _D), jnp.float32),
        ],
    )(hws, srcp, dstp, zeros128)


_PREC = lax.Precision.HIGHEST


def _dot(a, b):
    return jax.lax.dot_general(a, b, (((a.ndim - 1,), (0,)), ((), ())),
                               precision=_PREC,
                               preferred_element_type=jnp.float32)


def _dinv(d0_ref, d1_ref):
    return 1.0 / jnp.sqrt(d0_ref[...] + d1_ref[...] + 1.0)


def _mm_scale_body(x_ref, w_ref, d0_ref, d1_ref, o_ref):
    o_ref[...] = _dot(x_ref[...], w_ref[...]) * _dinv(d0_ref, d1_ref)


def _mm_scale(x, w, dg):
    return pl.pallas_call(
        _mm_scale_body,
        grid=(_NP // _BM,),
        in_specs=[
            pl.BlockSpec((_BM, _D), lambda i: (i, 0)),
            pl.BlockSpec((_D, _D), lambda i: (0, 0)),
            pl.BlockSpec((_BM, 1), lambda i: (i, 0)),
            pl.BlockSpec((_BM, 1), lambda i: (i + _NP // _BM, 0)),
        ],
        out_specs=pl.BlockSpec((_BM, _D), lambda i: (i, 0)),
        out_shape=jax.ShapeDtypeStruct((_NP, _D), jnp.float32),
    )(x, w, dg, dg)


def _stats_phase(a0_ref, a1_ref, hws_ref, d0_ref, d1_ref, b_ref,
                 s_scr, st_scr, i):
    s = (a0_ref[...] + a1_ref[...] + hws_ref[...]) * _dinv(d0_ref, d1_ref) \
        + b_ref[...]
    s_scr[...] = s
    rows = lax.broadcasted_iota(jnp.int32, (_BM, 1), 0) + i * _BM
    sv = jnp.where(rows < _N, s, 0.0)
    upd = jnp.concatenate(
        [jnp.sum(sv, axis=0, keepdims=True),
         jnp.sum(sv * sv, axis=0, keepdims=True),
         jnp.zeros((6, _D), jnp.float32)], axis=0)

    @pl.when(i == 0)
    def _():
        st_scr[...] = upd

    @pl.when(i > 0)
    def _():
        st_scr[...] += upd


def _bn(s, st_ref, g_ref, be_ref):
    m = st_ref[0:1, :] / _N
    var = st_ref[1:2, :] / _N - m * m
    rstd = 1.0 / jnp.sqrt(var + 1e-5)
    return jnp.maximum((s - m) * rstd * g_ref[...] + be_ref[...], 0.0)


def _layer_mm_body(a0_ref, a1_ref, hws_ref, d0_ref, d1_ref, b_ref,
                   g_ref, be_ref, w_ref, o_ref, s_scr, st_scr):
    p = pl.program_id(0)
    i = pl.program_id(1)

    @pl.when(p == 0)
    def _():
        _stats_phase(a0_ref, a1_ref, hws_ref, d0_ref, d1_ref, b_ref,
                     s_scr.at[i], st_scr, i)

    @pl.when(p == 1)
    def _():
        h = _bn(s_scr[i], st_scr, g_ref, be_ref)
        o_ref[...] = _dot(h, w_ref[...]) * _dinv(d0_ref, d1_ref)


def _layer_mm(ap, hws, dg, b, g, be, w):
    nb = _NP // _BM
    return pl.pallas_call(
        _layer_mm_body,
        grid=(2, nb),
        in_specs=[
            pl.BlockSpec((_BM, _D), lambda p, i: (i, 0)),
            pl.BlockSpec((_BM, _D), lambda p, i: (i + _NP // _BM, 0)),
            pl.BlockSpec((_BM, _D), lambda p, i: (i, 0)),
            pl.BlockSpec((_BM, 1), lambda p, i: (i, 0)),
            pl.BlockSpec((_BM, 1), lambda p, i: (i + _NP // _BM, 0)),
            pl.BlockSpec((1, _D), lambda p, i: (0, 0)),
            pl.BlockSpec((1, _D), lambda p, i: (0, 0)),
            pl.BlockSpec((1, _D), lambda p, i: (0, 0)),
            pl.BlockSpec((_D, _D), lambda p, i: (0, 0)),
        ],
        out_specs=pl.BlockSpec((_BM, _D), lambda p, i: (i, 0)),
        out_shape=jax.ShapeDtypeStruct((_NP, _D), jnp.float32),
        scratch_shapes=[
            pltpu.VMEM((_NP // _BM, _BM, _D), jnp.float32),
            pltpu.VMEM((8, _D), jnp.float32),
        ],
    )(ap, ap, hws, dg, dg, b, g, be, w)


def _layer_pool_body(a0_ref, a1_ref, hws_ref, d0_ref, d1_ref, b_ref,
                     g_ref, be_ref, batch_ref,
                     psum_ref, pmax_ref, pcnt_ref, s_scr, st_scr):
    p = pl.program_id(0)
    i = pl.program_id(1)

    @pl.when(p == 0)
    def _():
        _stats_phase(a0_ref, a1_ref, hws_ref, d0_ref, d1_ref, b_ref,
                     s_scr.at[i], st_scr, i)

    @pl.when(p == 1)
    def _():
        h = _bn(s_scr[i], st_scr, g_ref, be_ref)
        bids = batch_ref[...]
        oh = (bids == lax.broadcasted_iota(jnp.int32, (_BM, _B), 1))
        ohf = oh.astype(jnp.float32)
        dn = (((0,), (0,)), ((), ()))
        psum_upd = lax.dot_general(ohf, h, dn, precision=_PREC,
                                   preferred_element_type=jnp.float32)
        pcnt_upd = lax.dot_general(ohf, jnp.ones((_BM, _D), jnp.float32),
                                   dn, precision=_PREC,
                                   preferred_element_type=jnp.float32)
        neg = jnp.float32(-jnp.inf)
        pmax_upd = jnp.concatenate(
            [jnp.max(jnp.where(bids == b, h, neg), axis=0, keepdims=True)
             for b in range(_B)], axis=0)

        @pl.when(i == 0)
        def _():
            psum_ref[...] = psum_upd
            pmax_ref[...] = pmax_upd
            pcnt_ref[...] = pcnt_upd

        @pl.when(i > 0)
        def _():
            psum_ref[...] += psum_upd
            pmax_ref[...] = jnp.maximum(pmax_ref[...], pmax_upd)
            pcnt_ref[...] += pcnt_upd


def _layer_pool(ap, hws, dg, b, g, be, batch2d):
    nb = _NP // _BM
    return pl.pallas_call(
        _layer_pool_body,
        grid=(2, nb),
        in_specs=[
            pl.BlockSpec((_BM, _D), lambda p, i: (i, 0)),
            pl.BlockSpec((_BM, _D), lambda p, i: (i + _NP // _BM, 0)),
            pl.BlockSpec((_BM, _D), lambda p, i: (i, 0)),
            pl.BlockSpec((_BM, 1), lambda p, i: (i, 0)),
            pl.BlockSpec((_BM, 1), lambda p, i: (i + _NP // _BM, 0)),
            pl.BlockSpec((1, _D), lambda p, i: (0, 0)),
            pl.BlockSpec((1, _D), lambda p, i: (0, 0)),
            pl.BlockSpec((1, _D), lambda p, i: (0, 0)),
            pl.BlockSpec((_BM, 1), lambda p, i: (i, 0)),
        ],
        out_specs=[
            pl.BlockSpec((_B, _D), lambda p, i: (0, 0)),
            pl.BlockSpec((_B, _D), lambda p, i: (0, 0)),
            pl.BlockSpec((_B, _D), lambda p, i: (0, 0)),
        ],
        out_shape=[
            jax.ShapeDtypeStruct((_B, _D), jnp.float32),
            jax.ShapeDtypeStruct((_B, _D), jnp.float32),
            jax.ShapeDtypeStruct((_B, _D), jnp.float32),
        ],
        scratch_shapes=[
            pltpu.VMEM((_NP // _BM, _BM, _D), jnp.float32),
            pltpu.VMEM((8, _D), jnp.float32),
        ],
    )(ap, ap, hws, dg, dg, b, g, be, batch2d)


def _head_body(psum_ref, pmax_ref, pcnt_ref, w1_ref, b1_ref, w2_ref, b2_ref,
               w3_ref, b3_ref, o_ref):
    cnt = pcnt_ref[...]
    mean = psum_ref[...] / jnp.maximum(cnt, 1.0)
    mx = jnp.where(cnt > 0, pmax_ref[...], 0.0)
    z = jnp.concatenate([mean, mx], axis=1)
    z = jnp.maximum(_dot(z, w1_ref[...]) + b1_ref[...], 0.0)
    z = jnp.maximum(_dot(z, w2_ref[...]) + b2_ref[...], 0.0)
    z = _dot(z, w3_ref[...]) + b3_ref[...]
    o_ref[...] = 1.0 / (1.0 + jnp.exp(-z))


def _head(psum, pmax, pcnt, w1, b1, w2, b2, w3, b3):
    return pl.pallas_call(
        _head_body,
        out_shape=jax.ShapeDtypeStruct((_B, 1), jnp.float32),
    )(psum, pmax, pcnt, w1, b1, w2, b2, w3, b3)


def kernel(x, edge_index, batch, W0, b0, g0, be0, W1, b1, g1, be1,
           W2, b2, g2, be2, cW1, cb1, cW2, cb2, cW3, cb3):
    srcp = jnp.pad(edge_index[0], (0, _K))
    dstp = jnp.pad(edge_index[1], (0, _K))
    x_p = jnp.pad(x, ((0, _NP - _N), (0, 0)))
    batch_p = jnp.pad(batch, (0, _NP - _N),
                      constant_values=_B).reshape(_NP, 1)
    zeros128 = jnp.zeros((_RT, _D), jnp.float32)
    ones_rows = jnp.ones((_K, _D), jnp.float32)

    dg_raw = _deg_call(dstp, ones_rows, zeros128)
    dg = dg_raw[:, 0:1]

    hws = _mm_scale(x_p, W0, dg)
    layers = [(b0, g0, be0, W1), (b1, g1, be1, W2), (b2, g2, be2, None)]
    psum = pmax = pcnt = None
    for b, g, be, w_next in layers:
        ap = _scatter_call(hws, srcp, dstp, zeros128)
        if w_next is not None:
            hws = _layer_mm(ap, hws, dg, b.reshape(1, _D), g.reshape(1, _D),
                            be.reshape(1, _D), w_next)
        else:
            psum, pmax, pcnt = _layer_pool(ap, hws, dg, b.reshape(1, _D),
                                           g.reshape(1, _D),
                                           be.reshape(1, _D), batch_p)

    out = _head(psum, pmax, pcnt, cW1, cb1.reshape(1, _D),
                cW2, cb2.reshape(1, _D // 2), cW3, cb3.reshape(1, 1))
    return out.reshape(_B)
